# Initial kernel scaffold; baseline (speedup 1.0000x reference)
import jax
import jax.numpy as jnp
from jax.experimental import pallas as pl


def kernel(species, structure_ids, targets, weights):
    raise NotImplementedError("write your pallas kernel here")


# speedup vs baseline: 54.8403x; 54.8403x over previous
import functools

import jax
import jax.numpy as jnp
from jax import lax
from jax.experimental import pallas as pl
from jax.experimental.pallas import tpu as pltpu
from jax.experimental.pallas import tpu_sc as plsc

N_ATOMS = 3_200_000
N_STRUCT = 100_000
N_SPECIES = 16

LANES = 16
NW = 32
ROW = 128
ROWS_PAD = 25_600
ROWS_PER_W = ROWS_PAD // NW
CHUNK_ROWS = 16
N_CHUNKS = ROWS_PER_W // CHUNK_ROWS
PAD_ATOMS = ROWS_PAD * ROW - N_ATOMS

ACC_PAD = 100_352
ZSLICE = ACC_PAD // 16
CSLICE = ACC_PAD // NW


def _mesh():
    return plsc.VectorSubcoreMesh(core_axis_name="c", subcore_axis_name="s")


_SC_PARAMS = pltpu.CompilerParams(needs_layout_passes=False)


@functools.partial(
    pl.kernel,
    mesh=_mesh(),
    out_type=jax.ShapeDtypeStruct((2 * ACC_PAD,), jnp.float32),
    scratch_types=[
        pltpu.VMEM((N_SPECIES,), jnp.float32),
        pltpu.VMEM((CHUNK_ROWS, ROW), jnp.int32),
        pltpu.VMEM((CHUNK_ROWS, ROW), jnp.int32),
        pltpu.VMEM((CHUNK_ROWS, ROW), jnp.float32),
        pltpu.VMEM((ZSLICE,), jnp.float32),
        pltpu.VMEM_SHARED((ACC_PAD,), jnp.float32),
        pltpu.SemaphoreType.DMA,
    ],
    compiler_params=_SC_PARAMS,
)
def _partial_sums(sid_hbm, sp_hbm, w_hbm, p_hbm, wtab, sidbuf, spbuf, valbuf,
                  iobuf, acc, sem):
    c = lax.axis_index("c")
    s = lax.axis_index("s")
    wid = s * 2 + c

    pltpu.sync_copy(w_hbm, wtab)

    def zero_body(i, _):
        iobuf[pl.ds(i * LANES, LANES)] = jnp.zeros((LANES,), jnp.float32)
        return _
    lax.fori_loop(0, ZSLICE // LANES, zero_body, None)
    pltpu.sync_copy(iobuf, acc.at[pl.ds(s * ZSLICE, ZSLICE)])
    plsc.subcore_barrier()

    def chunk_body(t, _):
        base = wid * ROWS_PER_W + t * CHUNK_ROWS
        pltpu.sync_copy(sid_hbm.at[pl.ds(base, CHUNK_ROWS)], sidbuf)
        pltpu.sync_copy(sp_hbm.at[pl.ds(base, CHUNK_ROWS)], spbuf)

        copies = []
        for j in range(CHUNK_ROWS):
            for v in range(ROW // LANES):
                sp16 = spbuf[j, pl.ds(v * LANES, LANES)]
                valbuf[j, pl.ds(v * LANES, LANES)] = plsc.load_gather(
                    wtab, [sp16])
            copies.append(pltpu.async_copy(
                valbuf.at[j], acc.at[sidbuf.at[j]], sem, add=True))
        for cp in copies:
            cp.wait()
        return _
    lax.fori_loop(0, N_CHUNKS, chunk_body, None)

    plsc.subcore_barrier()
    pltpu.sync_copy(acc.at[pl.ds(s * ZSLICE, ZSLICE)], iobuf)
    pltpu.sync_copy(iobuf, p_hbm.at[pl.ds(c * ACC_PAD + s * ZSLICE, ZSLICE)])


@functools.partial(
    pl.kernel,
    mesh=_mesh(),
    out_type=jax.ShapeDtypeStruct((ACC_PAD,), jnp.float32),
    scratch_types=[
        pltpu.VMEM((CSLICE,), jnp.float32),
        pltpu.VMEM((CSLICE,), jnp.float32),
        pltpu.VMEM((CSLICE,), jnp.float32),
    ],
    compiler_params=_SC_PARAMS,
)
def _combine(p_hbm, t_hbm, out_hbm, b0, b1, bt):
    c = lax.axis_index("c")
    s = lax.axis_index("s")
    wid = s * 2 + c
    off = wid * CSLICE
    pltpu.sync_copy(p_hbm.at[pl.ds(off, CSLICE)], b0)
    pltpu.sync_copy(p_hbm.at[pl.ds(ACC_PAD + off, CSLICE)], b1)
    pltpu.sync_copy(t_hbm.at[pl.ds(off, CSLICE)], bt)

    def body(i, _):
        d = pl.ds(i * LANES, LANES)
        bt[d] = bt[d] - b0[d] - b1[d]
        return _
    lax.fori_loop(0, CSLICE // LANES, body, None)
    pltpu.sync_copy(bt, out_hbm.at[pl.ds(off, CSLICE)])


def kernel(species, structure_ids, targets, weights):
    pad_sid = N_STRUCT + (jnp.arange(PAD_ATOMS, dtype=jnp.int32)
                          % (ACC_PAD - N_STRUCT))
    sid2d = jnp.concatenate([structure_ids, pad_sid]).reshape(ROWS_PAD, ROW)
    sp2d = jnp.concatenate(
        [species, jnp.zeros((PAD_ATOMS,), jnp.int32)]).reshape(ROWS_PAD, ROW)
    t_pad = jnp.concatenate(
        [targets.reshape(-1), jnp.zeros((ACC_PAD - N_STRUCT,), jnp.float32)])
    w1 = weights.reshape(N_SPECIES)

    partials = _partial_sums(sid2d, sp2d, w1)
    out = _combine(partials, t_pad)
    return out[:N_STRUCT].reshape(N_STRUCT, 1)


# speedup vs baseline: 86.3105x; 1.5739x over previous
import functools

import jax
import jax.numpy as jnp
from jax import lax
from jax.experimental import pallas as pl
from jax.experimental.pallas import tpu as pltpu
from jax.experimental.pallas import tpu_sc as plsc

N_ATOMS = 3_200_000
N_STRUCT = 100_000
N_SPECIES = 16

LANES = 16
NW = 32
ROW = 128
ROWS_PAD = 25_600
ROWS_PER_W = ROWS_PAD // NW
CHUNK_ROWS = 40
N_CHUNKS = ROWS_PER_W // CHUNK_ROWS
NSLOT = 4
PAD_ATOMS = ROWS_PAD * ROW - N_ATOMS

ACC_PAD = 100_352
ZSLICE = ACC_PAD // 16
CSLICE = ACC_PAD // NW


def _mesh():
    return plsc.VectorSubcoreMesh(core_axis_name="c", subcore_axis_name="s")


_SC_PARAMS = pltpu.CompilerParams(needs_layout_passes=False)


@functools.partial(
    pl.kernel,
    mesh=_mesh(),
    out_type=jax.ShapeDtypeStruct((2 * ACC_PAD,), jnp.float32),
    scratch_types=[
        pltpu.VMEM((N_SPECIES,), jnp.float32),
        pltpu.VMEM((NSLOT, CHUNK_ROWS, ROW), jnp.int32),
        pltpu.VMEM((NSLOT, CHUNK_ROWS, ROW), jnp.int32),
        pltpu.VMEM((NSLOT, CHUNK_ROWS, ROW), jnp.float32),
        pltpu.VMEM((ZSLICE,), jnp.float32),
        pltpu.VMEM_SHARED((ACC_PAD,), jnp.float32),
        pltpu.SemaphoreType.DMA,
        pltpu.SemaphoreType.DMA,
        pltpu.SemaphoreType.DMA,
        pltpu.SemaphoreType.DMA,
        pltpu.SemaphoreType.DMA,
        pltpu.SemaphoreType.DMA,
        pltpu.SemaphoreType.DMA,
        pltpu.SemaphoreType.DMA,
    ],
    compiler_params=_SC_PARAMS,
)
def _partial_sums(sid_hbm, sp_hbm, w_hbm, p_hbm, wtab, sidb, spb, valb,
                  iobuf, acc, si0, si1, si2, si3, ss0, ss1, ss2, ss3):
    c = lax.axis_index("c")
    s = lax.axis_index("s")
    wid = s * 2 + c
    in_sems = [si0, si1, si2, si3]
    sc_sems = [ss0, ss1, ss2, ss3]

    def fire_in(t, j):
        base = wid * ROWS_PER_W + t * CHUNK_ROWS
        pltpu.async_copy(sid_hbm.at[pl.ds(base, CHUNK_ROWS)], sidb.at[j],
                         in_sems[j])
        pltpu.async_copy(sp_hbm.at[pl.ds(base, CHUNK_ROWS)], spb.at[j],
                         in_sems[j])

    def wait_in(j):
        pltpu.make_async_copy(sid_hbm.at[pl.ds(0, CHUNK_ROWS)], sidb.at[j],
                              in_sems[j]).wait()
        pltpu.make_async_copy(sp_hbm.at[pl.ds(0, CHUNK_ROWS)], spb.at[j],
                              in_sems[j]).wait()

    def compute(j):
        def row_body(r, _):
            for v in range(ROW // LANES):
                sp16 = spb[j, r, pl.ds(v * LANES, LANES)]
                valb[j, r, pl.ds(v * LANES, LANES)] = plsc.load_gather(
                    wtab, [sp16])
            return _
        lax.fori_loop(0, CHUNK_ROWS, row_body, None)

    def fire_scat(j):
        for r in range(CHUNK_ROWS):
            pltpu.async_copy(valb.at[j, r], acc.at[sidb.at[j, r]], sc_sems[j],
                             add=True)

    def drain_scat(j):
        for r in range(CHUNK_ROWS):
            pltpu.make_async_copy(valb.at[j, r], acc.at[sidb.at[j, r]],
                                  sc_sems[j]).wait()

    pltpu.sync_copy(w_hbm, wtab)
    fire_in(0, 0)
    fire_in(1, 1)

    def zero_body(i, _):
        iobuf[pl.ds(i * LANES, LANES)] = jnp.zeros((LANES,), jnp.float32)
        return _
    lax.fori_loop(0, ZSLICE // LANES, zero_body, None)
    pltpu.sync_copy(iobuf, acc.at[pl.ds(s * ZSLICE, ZSLICE)])
    plsc.subcore_barrier()

    def visit(t, j, do_drain, do_fire):
        wait_in(j)
        compute(j)
        fire_scat(j)
        j3 = (j + 2) % NSLOT
        if do_drain:
            drain_scat(j3)
        if do_fire:
            fire_in(t + 2, j3)

    for j in range(NSLOT):
        visit(j, j, do_drain=(j >= 2), do_fire=True)

    def steady_body(k, _):
        for j in range(NSLOT):
            visit(k * NSLOT + j, j, do_drain=True, do_fire=True)
        return _
    lax.fori_loop(1, N_CHUNKS // NSLOT - 1, steady_body, None)

    for j in range(NSLOT):
        visit(N_CHUNKS - NSLOT + j, j, do_drain=True, do_fire=(j < 2))
    drain_scat(2)
    drain_scat(3)

    plsc.subcore_barrier()
    pltpu.sync_copy(acc.at[pl.ds(s * ZSLICE, ZSLICE)], iobuf)
    pltpu.sync_copy(iobuf, p_hbm.at[pl.ds(c * ACC_PAD + s * ZSLICE, ZSLICE)])


@functools.partial(
    pl.kernel,
    mesh=_mesh(),
    out_type=jax.ShapeDtypeStruct((ACC_PAD,), jnp.float32),
    scratch_types=[
        pltpu.VMEM((CSLICE,), jnp.float32),
        pltpu.VMEM((CSLICE,), jnp.float32),
        pltpu.VMEM((CSLICE,), jnp.float32),
    ],
    compiler_params=_SC_PARAMS,
)
def _combine(p_hbm, t_hbm, out_hbm, b0, b1, bt):
    c = lax.axis_index("c")
    s = lax.axis_index("s")
    wid = s * 2 + c
    off = wid * CSLICE
    pltpu.sync_copy(p_hbm.at[pl.ds(off, CSLICE)], b0)
    pltpu.sync_copy(p_hbm.at[pl.ds(ACC_PAD + off, CSLICE)], b1)
    pltpu.sync_copy(t_hbm.at[pl.ds(off, CSLICE)], bt)

    def body(i, _):
        d = pl.ds(i * LANES, LANES)
        bt[d] = bt[d] - b0[d] - b1[d]
        return _
    lax.fori_loop(0, CSLICE // LANES, body, None)
    pltpu.sync_copy(bt, out_hbm.at[pl.ds(off, CSLICE)])


def kernel(species, structure_ids, targets, weights):
    pad_sid = N_STRUCT + (jnp.arange(PAD_ATOMS, dtype=jnp.int32)
                          % (ACC_PAD - N_STRUCT))
    sid2d = jnp.concatenate([structure_ids, pad_sid]).reshape(ROWS_PAD, ROW)
    sp2d = jnp.concatenate(
        [species, jnp.zeros((PAD_ATOMS,), jnp.int32)]).reshape(ROWS_PAD, ROW)
    t_pad = jnp.concatenate(
        [targets.reshape(-1), jnp.zeros((ACC_PAD - N_STRUCT,), jnp.float32)])
    w1 = weights.reshape(N_SPECIES)

    partials = _partial_sums(sid2d, sp2d, w1)
    out = _combine(partials, t_pad)
    return out[:N_STRUCT].reshape(N_STRUCT, 1)


# speedup vs baseline: 96.1945x; 1.1145x over previous
import functools

import jax
import jax.numpy as jnp
from jax import lax
from jax.experimental import pallas as pl
from jax.experimental.pallas import tpu as pltpu
from jax.experimental.pallas import tpu_sc as plsc

N_ATOMS = 3_200_000
N_STRUCT = 100_000
N_SPECIES = 16

LANES = 16
NW = 32
ROW = 128
ROWS = N_ATOMS // ROW
CHUNK_ROWS = 40
TOT_CHUNKS = ROWS // CHUNK_ROWS
N_CHUNKS = 20
FULL_W = TOT_CHUNKS - (N_CHUNKS - 1) * NW
NSLOT = 4

ACC_PAD = 100_352
ZSLICE = ACC_PAD // 16
CSLICE = ACC_PAD // NW


def _mesh():
    return plsc.VectorSubcoreMesh(core_axis_name="c", subcore_axis_name="s")


_SC_PARAMS = pltpu.CompilerParams(needs_layout_passes=False)


@functools.partial(
    pl.kernel,
    mesh=_mesh(),
    out_type=jax.ShapeDtypeStruct((2 * ACC_PAD,), jnp.float32),
    scratch_types=[
        pltpu.VMEM((N_SPECIES,), jnp.float32),
        pltpu.VMEM((NSLOT, CHUNK_ROWS, ROW), jnp.int32),
        pltpu.VMEM((NSLOT, CHUNK_ROWS, ROW), jnp.int32),
        pltpu.VMEM((NSLOT, CHUNK_ROWS, ROW), jnp.float32),
        pltpu.VMEM((ZSLICE,), jnp.float32),
        pltpu.VMEM_SHARED((ACC_PAD,), jnp.float32),
        pltpu.SemaphoreType.DMA,
        pltpu.SemaphoreType.DMA,
        pltpu.SemaphoreType.DMA,
        pltpu.SemaphoreType.DMA,
        pltpu.SemaphoreType.DMA,
        pltpu.SemaphoreType.DMA,
        pltpu.SemaphoreType.DMA,
        pltpu.SemaphoreType.DMA,
    ],
    compiler_params=_SC_PARAMS,
)
def _partial_sums(sid_hbm, sp_hbm, w_hbm, p_hbm, wtab, sidb, spb, valb,
                  iobuf, acc, si0, si1, si2, si3, ss0, ss1, ss2, ss3):
    c = lax.axis_index("c")
    s = lax.axis_index("s")
    wid = s * 2 + c
    in_sems = [si0, si1, si2, si3]
    sc_sems = [ss0, ss1, ss2, ss3]

    def fire_in(t, j):
        base = jnp.minimum(wid + t * NW, TOT_CHUNKS - 1) * CHUNK_ROWS
        pltpu.async_copy(sid_hbm.at[pl.ds(base, CHUNK_ROWS)], sidb.at[j],
                         in_sems[j])
        pltpu.async_copy(sp_hbm.at[pl.ds(base, CHUNK_ROWS)], spb.at[j],
                         in_sems[j])

    def wait_in(j):
        pltpu.make_async_copy(sid_hbm.at[pl.ds(0, CHUNK_ROWS)], sidb.at[j],
                              in_sems[j]).wait()
        pltpu.make_async_copy(sp_hbm.at[pl.ds(0, CHUNK_ROWS)], spb.at[j],
                              in_sems[j]).wait()

    def compute(t, j):
        m = jnp.where(jnp.logical_and(wid >= FULL_W, t == N_CHUNKS - 1),
                      jnp.float32(0.0), jnp.float32(1.0))

        def row_body(r, _):
            for v in range(ROW // LANES):
                sp16 = spb[j, r, pl.ds(v * LANES, LANES)]
                valb[j, r, pl.ds(v * LANES, LANES)] = m * plsc.load_gather(
                    wtab, [sp16])
            return _
        lax.fori_loop(0, CHUNK_ROWS, row_body, None)

    def fire_scat(j):
        for r in range(CHUNK_ROWS):
            pltpu.async_copy(valb.at[j, r], acc.at[sidb.at[j, r]], sc_sems[j],
                             add=True)

    def drain_scat(j):
        for r in range(CHUNK_ROWS):
            pltpu.make_async_copy(valb.at[j, r], acc.at[sidb.at[j, r]],
                                  sc_sems[j]).wait()

    pltpu.sync_copy(w_hbm, wtab)
    fire_in(0, 0)
    fire_in(1, 1)

    def zero_body(i, _):
        iobuf[pl.ds(i * LANES, LANES)] = jnp.zeros((LANES,), jnp.float32)
        return _
    lax.fori_loop(0, ZSLICE // LANES, zero_body, None)
    pltpu.sync_copy(iobuf, acc.at[pl.ds(s * ZSLICE, ZSLICE)])
    plsc.subcore_barrier()

    def visit(t, j, do_drain, do_fire):
        wait_in(j)
        compute(t, j)
        fire_scat(j)
        j3 = (j + 2) % NSLOT
        if do_drain:
            drain_scat(j3)
        if do_fire:
            fire_in(t + 2, j3)

    for j in range(NSLOT):
        visit(j, j, do_drain=(j >= 2), do_fire=True)

    def steady_body(k, _):
        for j in range(NSLOT):
            visit(k * NSLOT + j, j, do_drain=True, do_fire=True)
        return _
    lax.fori_loop(1, N_CHUNKS // NSLOT - 1, steady_body, None)

    for j in range(NSLOT):
        visit(N_CHUNKS - NSLOT + j, j, do_drain=True, do_fire=(j < 2))
    drain_scat(2)
    drain_scat(3)

    plsc.subcore_barrier()
    pltpu.sync_copy(acc.at[pl.ds(s * ZSLICE, ZSLICE)], iobuf)
    pltpu.sync_copy(iobuf, p_hbm.at[pl.ds(c * ACC_PAD + s * ZSLICE, ZSLICE)])


@functools.partial(
    pl.kernel,
    mesh=_mesh(),
    out_type=jax.ShapeDtypeStruct((ACC_PAD,), jnp.float32),
    scratch_types=[
        pltpu.VMEM((CSLICE,), jnp.float32),
        pltpu.VMEM((CSLICE,), jnp.float32),
        pltpu.VMEM((CSLICE,), jnp.float32),
    ],
    compiler_params=_SC_PARAMS,
)
def _combine(p_hbm, t_hbm, out_hbm, b0, b1, bt):
    c = lax.axis_index("c")
    s = lax.axis_index("s")
    wid = s * 2 + c
    off = wid * CSLICE
    pltpu.sync_copy(p_hbm.at[pl.ds(off, CSLICE)], b0)
    pltpu.sync_copy(p_hbm.at[pl.ds(ACC_PAD + off, CSLICE)], b1)
    pltpu.sync_copy(t_hbm.at[pl.ds(off, CSLICE)], bt)

    def body(i, _):
        d = pl.ds(i * LANES, LANES)
        bt[d] = bt[d] - b0[d] - b1[d]
        return _
    lax.fori_loop(0, CSLICE // LANES, body, None)
    pltpu.sync_copy(bt, out_hbm.at[pl.ds(off, CSLICE)])


def kernel(species, structure_ids, targets, weights):
    sid2d = structure_ids.reshape(ROWS, ROW)
    sp2d = species.reshape(ROWS, ROW)
    t_pad = jnp.concatenate(
        [targets.reshape(-1), jnp.zeros((ACC_PAD - N_STRUCT,), jnp.float32)])
    w1 = weights.reshape(N_SPECIES)

    partials = _partial_sums(sid2d, sp2d, w1)
    out = _combine(partials, t_pad)
    return out[:N_STRUCT].reshape(N_STRUCT, 1)
